# baseline (speedup 1.0000x reference)
import functools

import jax
import jax.numpy as jnp
from jax import lax
from jax.experimental import pallas as pl
from jax.experimental.pallas import tpu as pltpu
from jax.experimental.pallas import tpu_sc as plsc

N_ITERS = 3
B = 256
D = 128
NC = 2
NS = 16
NW = NC * NS

W16 = 196
BLK16 = 2
NBLK = W16 // BLK16


def _sc_pass_body(feat_hbm, seg_hbm, q_hbm, z_hbm, z16_hbm, r_out, s_out,
                  q_v, seg_v, feat_v0, feat_v1, r_v, s_v, stage_v,
                  sem0, sem1, *, n16):
    wid = lax.axis_index("s") * NC + lax.axis_index("c")
    start16 = jnp.minimum(wid * W16, n16 - W16)
    skip16 = wid * W16 - start16

    pltpu.sync_copy(seg_hbm.at[pl.ds(start16 * 16, W16 * 16)], seg_v)
    pltpu.sync_copy(q_hbm, q_v)
    pltpu.sync_copy(z_hbm, r_v)
    pltpu.sync_copy(z16_hbm, s_v)

    def _feat_dma(blk, buf, sem):
        arow0 = (start16 + blk * BLK16) * 16
        return pltpu.make_async_copy(
            feat_hbm.at[pl.ds(arow0, BLK16 * 16), :], buf, sem)

    def _process(blk, feat_v):
        g_lo = jnp.clip(skip16 - blk * BLK16, 0, BLK16)

        def group_body(g, c2):
            sv = seg_v[pl.ds((blk * BLK16 + g) * 16, 16)]
            s0 = sv[0]
            zero16 = jnp.zeros((16,), jnp.float32)
            stage_v[...] = zero16

            def _dot_stage(qrow_of):
                acc = [None] * 16
                for k in range(8):
                    sl = pl.ds(k * 16, 16)
                    for l in range(16):
                        t = feat_v[g * 16 + l, sl] * qrow_of(l, k, sl)
                        acc[l] = t if k == 0 else acc[l] + t
                for l in range(16):
                    plsc.addupdate_scatter(
                        stage_v, [jnp.full((16,), l, jnp.int32)], acc[l])
                return jnp.exp(stage_v[...])

            @pl.when(s0 == sv[15])
            def _uniform():
                qk = [q_v[s0, pl.ds(k * 16, 16)] for k in range(8)]
                pvec = _dot_stage(lambda l, k, sl, _q=qk: _q[k])
                ra = [zero16] * 8
                rb = [zero16] * 8
                for l in range(16):
                    pb = pvec[l] + zero16
                    tgt = ra if l % 2 == 0 else rb
                    for k in range(8):
                        sl = pl.ds(k * 16, 16)
                        tgt[k] = tgt[k] + pb * feat_v[g * 16 + l, sl]
                for k in range(8):
                    plsc.addupdate(r_v.at[s0, pl.ds(k * 16, 16)],
                                   ra[k] + rb[k])
                plsc.addupdate(s_v.at[s0, :], pvec)

            @pl.when(s0 != sv[15])
            def _general():
                pvec = _dot_stage(lambda l, k, sl: q_v[sv[l], sl])
                for l in range(16):
                    row = g * 16 + l
                    sidx = sv[l]
                    pb = pvec[l] + zero16
                    for k in range(8):
                        sl = pl.ds(k * 16, 16)
                        plsc.addupdate(r_v.at[sidx, sl],
                                       pb * feat_v[row, sl])
                plsc.addupdate_scatter(
                    s_v, [sv, lax.iota(jnp.int32, 16)], pvec)
            return c2

        lax.fori_loop(g_lo, BLK16, group_body, jnp.int32(0))

    _feat_dma(0, feat_v0, sem0).start()

    def pair_body(i, carry):
        blk0 = i * 2
        _feat_dma(blk0 + 1, feat_v1, sem1).start()
        _feat_dma(blk0, feat_v0, sem0).wait()
        _process(blk0, feat_v0)

        @pl.when(blk0 + 2 < NBLK)
        def _prefetch():
            _feat_dma(blk0 + 2, feat_v0, sem0).start()

        _feat_dma(blk0 + 1, feat_v1, sem1).wait()
        _process(blk0 + 1, feat_v1)
        return carry

    lax.fori_loop(0, NBLK // 2, pair_body, jnp.int32(0))
    pltpu.sync_copy(r_v, r_out.at[wid])
    pltpu.sync_copy(s_v, s_out.at[wid])


def _sc_pass(feat, seg, q, zeros_bd, zeros_b16):
    n16 = seg.shape[0] // 16
    mesh = plsc.VectorSubcoreMesh(core_axis_name="c", subcore_axis_name="s")
    kern = pl.kernel(
        functools.partial(_sc_pass_body, n16=n16),
        mesh=mesh,
        compiler_params=pltpu.CompilerParams(needs_layout_passes=False),
        out_type=[
            jax.ShapeDtypeStruct((NW, B, D), jnp.float32),
            jax.ShapeDtypeStruct((NW, B, 16), jnp.float32),
        ],
        scratch_types=[
            pltpu.VMEM((B, D), jnp.float32),
            pltpu.VMEM((W16 * 16,), jnp.int32),
            pltpu.VMEM((BLK16 * 16, D), jnp.float32),
            pltpu.VMEM((BLK16 * 16, D), jnp.float32),
            pltpu.VMEM((B, D), jnp.float32),
            pltpu.VMEM((B, 16), jnp.float32),
            pltpu.VMEM((16,), jnp.float32),
            pltpu.SemaphoreType.DMA,
            pltpu.SemaphoreType.DMA,
        ],
    )
    return kern(feat, seg, q, zeros_bd, zeros_b16)


def _tc_step_body(rp_ref, sp_ref, qprev_ref, h_ref, c_ref, wih_ref, whh_ref,
                  b_ref, hn_ref, cn_ref, qn_ref, qs_ref, *, d):
    r = jnp.sum(rp_ref[...], axis=0)
    s = jnp.sum(jnp.sum(sp_ref[...], axis=0), axis=1, keepdims=True)
    readout = jnp.where(s > 0.0, r / s, 0.0)
    qprev = qprev_ref[...]
    qs = jnp.concatenate([qprev, readout], axis=1)
    qs_ref[...] = qs
    gates = (lax.dot(qs, wih_ref[...], preferred_element_type=jnp.float32,
                     precision=lax.Precision.HIGHEST)
             + lax.dot(h_ref[...], whh_ref[...],
                       preferred_element_type=jnp.float32,
                       precision=lax.Precision.HIGHEST)
             + b_ref[...])
    gi = gates[:, 0:d]
    gf = gates[:, d:2 * d]
    gg = gates[:, 2 * d:3 * d]
    go = gates[:, 3 * d:4 * d]
    c_new = jax.nn.sigmoid(gf) * c_ref[...] + jax.nn.sigmoid(gi) * jnp.tanh(gg)
    h_new = jax.nn.sigmoid(go) * jnp.tanh(c_new)
    hn_ref[...] = h_new
    cn_ref[...] = c_new
    qn_ref[...] = h_new


def _tc_step(rp, sp, qprev, h, c, wih_t, whh_t, bias):
    d = D
    return pl.pallas_call(
        functools.partial(_tc_step_body, d=d),
        out_shape=(
            jax.ShapeDtypeStruct((B, d), jnp.float32),
            jax.ShapeDtypeStruct((B, d), jnp.float32),
            jax.ShapeDtypeStruct((B, d), jnp.float32),
            jax.ShapeDtypeStruct((B, 2 * d), jnp.float32),
        ),
    )(rp, sp, qprev, h, c, wih_t, whh_t, bias)


def _set2set(feat, seg, w_ih, w_hh, b_ih, b_hh):
    n, d = feat.shape
    wih_t = w_ih.T
    whh_t = w_hh.T
    bias = (b_ih + b_hh).reshape(1, 4 * d)
    z_bd = jnp.zeros((B, d), jnp.float32)
    z_b16 = jnp.zeros((B, 16), jnp.float32)
    z_rp = jnp.zeros((NW, B, d), jnp.float32)
    z_sp = jnp.zeros((NW, B, 16), jnp.float32)
    h, c, q, _ = _tc_step(z_rp, z_sp, z_bd, z_bd, z_bd, wih_t, whh_t, bias)
    qstar = None
    for _ in range(N_ITERS):
        rp, sp = _sc_pass(feat, seg, q, z_bd, z_b16)
        h, c, q, qstar = _tc_step(rp, sp, q, h, c, wih_t, whh_t, bias)
    return qstar


def kernel(atom_feat, bond_feat, global_feat, atom_batch, bond_batch,
           atom_W_ih, atom_W_hh, atom_b_ih, atom_b_hh,
           bond_W_ih, bond_W_hh, bond_b_ih, bond_b_hh):
    a = _set2set(atom_feat, atom_batch, atom_W_ih, atom_W_hh, atom_b_ih,
                 atom_b_hh)
    b = _set2set(bond_feat, bond_batch, bond_W_ih, bond_W_hh, bond_b_ih,
                 bond_b_hh)
    return jnp.concatenate([a, b, global_feat], axis=-1)


# speedup vs baseline: 1.2359x; 1.2359x over previous
import functools

import jax
import jax.numpy as jnp
from jax import lax
from jax.experimental import pallas as pl
from jax.experimental.pallas import tpu as pltpu
from jax.experimental.pallas import tpu_sc as plsc

N_ITERS = 3
B = 256
D = 128
NC = 2
NS = 16
NW = NC * NS

W16 = 196
BLK16 = 2
NBLK = W16 // BLK16


def _sc_pass_body(feat_hbm, seg_hbm, q_hbm, z_hbm, z16_hbm, r_out, s_out,
                  q_v, seg_v, feat_v0, feat_v1, r_v, s_v, stage_v,
                  sem0, sem1, *, n16):
    wid = lax.axis_index("s") * NC + lax.axis_index("c")
    start16 = jnp.minimum(wid * W16, n16 - W16)
    skip16 = wid * W16 - start16

    pltpu.sync_copy(seg_hbm.at[pl.ds(start16 * 16, W16 * 16)], seg_v)
    pltpu.sync_copy(q_hbm, q_v)
    pltpu.sync_copy(z_hbm, r_v)
    pltpu.sync_copy(z16_hbm, s_v)

    def _feat_dma(blk, buf, sem):
        arow0 = (start16 + blk * BLK16) * 16
        return pltpu.make_async_copy(
            feat_hbm.at[pl.ds(arow0, BLK16 * 16), :], buf, sem)

    def _process(blk, feat_v):
        g_lo = jnp.clip(skip16 - blk * BLK16, 0, BLK16)

        def group_body(g, c2):
            sv = seg_v[pl.ds((blk * BLK16 + g) * 16, 16)]
            s0 = sv[0]
            zero16 = jnp.zeros((16,), jnp.float32)
            stage_v[...] = zero16

            def _dot_stage(qrow_of):
                acc = [None] * 16
                for k in range(8):
                    sl = pl.ds(k * 16, 16)
                    for l in range(16):
                        t = feat_v[g * 16 + l, sl] * qrow_of(l, k, sl)
                        acc[l] = t if k == 0 else acc[l] + t
                for l in range(16):
                    plsc.addupdate_scatter(
                        stage_v, [jnp.full((16,), l, jnp.int32)], acc[l])
                return jnp.exp(stage_v[...])

            @pl.when(s0 == sv[15])
            def _uniform():
                qk = [q_v[s0, pl.ds(k * 16, 16)] for k in range(8)]
                pvec = _dot_stage(lambda l, k, sl, _q=qk: _q[k])
                ra = [zero16] * 8
                rb = [zero16] * 8
                for l in range(16):
                    pb = pvec[l] + zero16
                    tgt = ra if l % 2 == 0 else rb
                    for k in range(8):
                        sl = pl.ds(k * 16, 16)
                        tgt[k] = tgt[k] + pb * feat_v[g * 16 + l, sl]
                for k in range(8):
                    plsc.addupdate(r_v.at[s0, pl.ds(k * 16, 16)],
                                   ra[k] + rb[k])
                plsc.addupdate(s_v.at[s0, :], pvec)

            @pl.when(s0 != sv[15])
            def _general():
                pvec = _dot_stage(lambda l, k, sl: q_v[sv[l], sl])
                for l in range(16):
                    row = g * 16 + l
                    sidx = sv[l]
                    pb = pvec[l] + zero16
                    for k in range(8):
                        sl = pl.ds(k * 16, 16)
                        plsc.addupdate(r_v.at[sidx, sl],
                                       pb * feat_v[row, sl])
                plsc.addupdate_scatter(
                    s_v, [sv, lax.iota(jnp.int32, 16)], pvec)
            return c2

        lax.fori_loop(g_lo, BLK16, group_body, jnp.int32(0))

    _feat_dma(0, feat_v0, sem0).start()

    def pair_body(i, carry):
        blk0 = i * 2
        _feat_dma(blk0 + 1, feat_v1, sem1).start()
        _feat_dma(blk0, feat_v0, sem0).wait()
        _process(blk0, feat_v0)

        @pl.when(blk0 + 2 < NBLK)
        def _prefetch():
            _feat_dma(blk0 + 2, feat_v0, sem0).start()

        _feat_dma(blk0 + 1, feat_v1, sem1).wait()
        _process(blk0 + 1, feat_v1)
        return carry

    lax.fori_loop(0, NBLK // 2, pair_body, jnp.int32(0))
    pltpu.sync_copy(r_v, r_out.at[wid])
    pltpu.sync_copy(s_v, s_out.at[wid])


def _sc_pass(feat, seg, q, zeros_bd, zeros_b16):
    n16 = seg.shape[0] // 16
    mesh = plsc.VectorSubcoreMesh(core_axis_name="c", subcore_axis_name="s")
    kern = pl.kernel(
        functools.partial(_sc_pass_body, n16=n16),
        mesh=mesh,
        compiler_params=pltpu.CompilerParams(needs_layout_passes=False),
        out_type=[
            jax.ShapeDtypeStruct((NW, B, D), jnp.float32),
            jax.ShapeDtypeStruct((NW, B, 16), jnp.float32),
        ],
        scratch_types=[
            pltpu.VMEM((B, D), jnp.float32),
            pltpu.VMEM((W16 * 16,), jnp.int32),
            pltpu.VMEM((BLK16 * 16, D), jnp.float32),
            pltpu.VMEM((BLK16 * 16, D), jnp.float32),
            pltpu.VMEM((B, D), jnp.float32),
            pltpu.VMEM((B, 16), jnp.float32),
            pltpu.VMEM((16,), jnp.float32),
            pltpu.SemaphoreType.DMA,
            pltpu.SemaphoreType.DMA,
        ],
    )
    return kern(feat, seg, q, zeros_bd, zeros_b16)


def _tc_step_body(rp_ref, sp_ref, qprev_ref, h_ref, c_ref, wih_ref, whh_ref,
                  b_ref, hn_ref, cn_ref, qn_ref, qs_ref, *, d):
    r = jnp.sum(rp_ref[...], axis=0)
    s = jnp.sum(jnp.sum(sp_ref[...], axis=0), axis=1, keepdims=True)
    readout = jnp.where(s > 0.0, r / s, 0.0)
    qprev = qprev_ref[...]
    qs = jnp.concatenate([qprev, readout], axis=1)
    qs_ref[...] = qs
    gates = (lax.dot(qs, wih_ref[...], preferred_element_type=jnp.float32,
                     precision=lax.Precision.HIGHEST)
             + lax.dot(h_ref[...], whh_ref[...],
                       preferred_element_type=jnp.float32,
                       precision=lax.Precision.HIGHEST)
             + b_ref[...])
    gi = gates[:, 0:d]
    gf = gates[:, d:2 * d]
    gg = gates[:, 2 * d:3 * d]
    go = gates[:, 3 * d:4 * d]
    c_new = jax.nn.sigmoid(gf) * c_ref[...] + jax.nn.sigmoid(gi) * jnp.tanh(gg)
    h_new = jax.nn.sigmoid(go) * jnp.tanh(c_new)
    hn_ref[...] = h_new
    cn_ref[...] = c_new
    qn_ref[...] = h_new


def _tc_step(rp, sp, qprev, h, c, wih_t, whh_t, bias):
    d = D
    return pl.pallas_call(
        functools.partial(_tc_step_body, d=d),
        out_shape=(
            jax.ShapeDtypeStruct((B, d), jnp.float32),
            jax.ShapeDtypeStruct((B, d), jnp.float32),
            jax.ShapeDtypeStruct((B, d), jnp.float32),
            jax.ShapeDtypeStruct((B, 2 * d), jnp.float32),
        ),
    )(rp, sp, qprev, h, c, wih_t, whh_t, bias)


CH = 2000


def _tc_s2s_body(seg_row_ref, feat_ref, wih_ref, whh_ref, b_ref,
              out_ref, h_ref, c_ref, q_ref, qh_ref, ql_ref, qs_ref, s_ref,
              r_ref, *, nchunk, ch, d):
    i = pl.program_id(0)
    j = pl.program_id(1)

    @pl.when(j == 0)
    def _start_iter():
        @pl.when(i == 0)
        def _init():
            h_ref[...] = jnp.zeros_like(h_ref)
            c_ref[...] = jnp.zeros_like(c_ref)
            qs_ref[...] = jnp.zeros_like(qs_ref)

        qs = qs_ref[...]
        h = h_ref[...]
        c = c_ref[...]
        gates = (lax.dot(qs, wih_ref[...], preferred_element_type=jnp.float32,
                         precision=lax.Precision.HIGHEST)
                 + lax.dot(h, whh_ref[...], preferred_element_type=jnp.float32,
                           precision=lax.Precision.HIGHEST)
                 + b_ref[...])
        gi = gates[:, 0:d]
        gf = gates[:, d:2 * d]
        gg = gates[:, 2 * d:3 * d]
        go = gates[:, 3 * d:4 * d]
        c_new = jax.nn.sigmoid(gf) * c + jax.nn.sigmoid(gi) * jnp.tanh(gg)
        h_new = jax.nn.sigmoid(go) * jnp.tanh(c_new)
        h_ref[...] = h_new
        c_ref[...] = c_new
        q_ref[...] = h_new
        qh = h_new.astype(jnp.bfloat16)
        qh_ref[...] = qh
        ql_ref[...] = (h_new - qh.astype(jnp.float32)).astype(jnp.bfloat16)
        s_ref[...] = jnp.zeros_like(s_ref)
        r_ref[...] = jnp.zeros_like(r_ref)

    feat = feat_ref[...]
    seg_r = seg_row_ref[0]
    maskT = (lax.broadcasted_iota(jnp.int32, (B, ch), 0)
             == seg_r).astype(jnp.bfloat16)
    tdims = (((0,), (0,)), ((), ()))
    qseg = (lax.dot_general(maskT, qh_ref[...], tdims,
                            preferred_element_type=jnp.float32)
            + lax.dot_general(maskT, ql_ref[...], tdims,
                              preferred_element_type=jnp.float32))
    e = jnp.sum(feat * qseg, axis=1, keepdims=True)
    p = jnp.exp(e)
    pf = (p * feat).astype(jnp.bfloat16)
    ph = p.astype(jnp.bfloat16)
    r_ref[...] += lax.dot(maskT, pf, preferred_element_type=jnp.float32)
    s_ref[...] += lax.dot(maskT, ph, preferred_element_type=jnp.float32)

    @pl.when(j == nchunk - 1)
    def _end_iter():
        s = s_ref[...]
        r = r_ref[...]
        readout = jnp.where(s > 0.0, r / s, 0.0)
        qs_ref[:, 0:d] = q_ref[...]
        qs_ref[:, d:2 * d] = readout

        @pl.when(i == N_ITERS - 1)
        def _write():
            out_ref[:, 0:d] = q_ref[...]
            out_ref[:, d:2 * d] = readout


def _set2set_tc(feat, seg, w_ih, w_hh, b_ih, b_hh):
    n, d = feat.shape
    ch = CH
    nchunk = n // ch
    seg_row = seg.reshape(nchunk, 1, ch)
    wih_t = w_ih.T
    whh_t = w_hh.T
    bias = (b_ih + b_hh).reshape(1, 4 * d)
    return pl.pallas_call(
        functools.partial(_tc_s2s_body, nchunk=nchunk, ch=ch, d=d),
        grid=(N_ITERS, nchunk),
        in_specs=[
            pl.BlockSpec((1, 1, ch), lambda i, j: (j, 0, 0)),
            pl.BlockSpec((ch, d), lambda i, j: (j, 0)),
            pl.BlockSpec((2 * d, 4 * d), lambda i, j: (0, 0)),
            pl.BlockSpec((d, 4 * d), lambda i, j: (0, 0)),
            pl.BlockSpec((1, 4 * d), lambda i, j: (0, 0)),
        ],
        out_specs=pl.BlockSpec((B, 2 * d), lambda i, j: (0, 0)),
        out_shape=jax.ShapeDtypeStruct((B, 2 * d), jnp.float32),
        scratch_shapes=[
            pltpu.VMEM((B, d), jnp.float32),
            pltpu.VMEM((B, d), jnp.float32),
            pltpu.VMEM((B, d), jnp.float32),
            pltpu.VMEM((B, d), jnp.bfloat16),
            pltpu.VMEM((B, d), jnp.bfloat16),
            pltpu.VMEM((B, 2 * d), jnp.float32),
            pltpu.VMEM((B, 1), jnp.float32),
            pltpu.VMEM((B, d), jnp.float32),
        ],
    )(seg_row, feat, wih_t, whh_t, bias)


def _set2set_sc(feat, seg, w_ih, w_hh, b_ih, b_hh):
    n, d = feat.shape
    wih_t = w_ih.T
    whh_t = w_hh.T
    bias = (b_ih + b_hh).reshape(1, 4 * d)
    z_bd = jnp.zeros((B, d), jnp.float32)
    z_b16 = jnp.zeros((B, 16), jnp.float32)
    z_rp = jnp.zeros((NW, B, d), jnp.float32)
    z_sp = jnp.zeros((NW, B, 16), jnp.float32)
    h, c, q, _ = _tc_step(z_rp, z_sp, z_bd, z_bd, z_bd, wih_t, whh_t, bias)
    qstar = None
    for _ in range(N_ITERS):
        rp, sp = _sc_pass(feat, seg, q, z_bd, z_b16)
        h, c, q, qstar = _tc_step(rp, sp, q, h, c, wih_t, whh_t, bias)
    return qstar


def kernel(atom_feat, bond_feat, global_feat, atom_batch, bond_batch,
           atom_W_ih, atom_W_hh, atom_b_ih, atom_b_hh,
           bond_W_ih, bond_W_hh, bond_b_ih, bond_b_hh):
    a = _set2set_sc(atom_feat, atom_batch, atom_W_ih, atom_W_hh, atom_b_ih,
                    atom_b_hh)
    b = _set2set_tc(bond_feat, bond_batch, bond_W_ih, bond_W_hh, bond_b_ih,
                    bond_b_hh)
    return jnp.concatenate([a, b, global_feat], axis=-1)
